# baseline (speedup 1.0000x reference)
import functools
import math

import numpy as np
import jax
import jax.numpy as jnp
from jax import lax
from jax.experimental import pallas as pl
from jax.experimental.pallas import tpu as pltpu
from jax.experimental.pallas import tpu_sc as plsc

_ALPHA, _BETA, _GAMMA = 2.0, 4.0, 8.0
_E = 24
_NSEG = 2 * _E - 1
_MW = _NSEG * _E
_MWP = 1136
_NC, _NS = 2, 16


def _pw_index(rp):
    rp = np.asarray(rp, dtype=np.float64)
    rp_abs = np.abs(rp)
    not_mask = rp_abs > _ALPHA
    idx = np.round(rp).astype(np.int64)
    rp_out = rp[not_mask]
    rp_abs_out = rp_abs[not_mask]
    y = (np.sign(rp_out) * np.clip(
        np.round(_ALPHA + np.log(rp_abs_out / _ALPHA)
                 / math.log(_GAMMA / _ALPHA) * (_BETA - _ALPHA)),
        None, _BETA)).astype(np.int64)
    idx[not_mask] = y
    return idx


def _quant(ids):
    uq, inv = np.unique(ids, return_inverse=True)
    return inv.reshape(ids.shape), uq.size


@functools.lru_cache(maxsize=None)
def _master_idx(L):
    E = int(math.isqrt(L))
    assert E * E == L and E == _E
    rg = np.arange(E)
    rows = np.repeat(rg[:, None], E, axis=1)
    cols = rows.T
    pos = np.stack([rows, cols], 2).reshape(E * E, 2)
    diff = pos[:, None, :] - pos[None, :, :]
    r, r_num = _quant(_pw_index(diff[:, :, 0]))
    c, c_num = _quant(_pw_index(diff[:, :, 1]))
    pid = (r * c_num + c).astype(np.int32)

    Rmap = np.zeros(_NSEG, np.int32)
    Cmap = np.zeros(_NSEG, np.int32)
    for d in range(-(E - 1), E):
        Rmap[d + E - 1] = r[max(d, 0) * E, max(-d, 0) * E]
        Cmap[d + E - 1] = c[max(d, 0), max(-d, 0)]

    ci = np.arange(E)
    seg = Cmap[(ci[:, None] - ci[None, :]) + E - 1]
    base = (Rmap[::-1] * c_num)
    idx = (base[None, :, None] + seg[:, None, :]).reshape(E, _MW)
    out = np.zeros((E, _MWP), np.int32)
    out[:, :_MW] = idx

    for ri in range(E):
        s = (E - 1 - ri) * E
        assert np.array_equal(idx[:, s:s + L], pid[ri * E:(ri + 1) * E, :])
    return out.reshape(-1)


def _sc_body(tab_hbm, idx_hbm, out_hbm, tab_v, idx_v, master_v, sem):
    c = lax.axis_index("c")
    s = lax.axis_index("s")
    w = s * _NC + c

    pltpu.sync_copy(tab_hbm, tab_v)
    pltpu.sync_copy(idx_hbm, idx_v)

    small = w < 24
    h = jnp.where(small, w // 3, 8 + (w - 24) // 2)
    rstart = jnp.where(small, (w % 3) * 8, ((w - 24) % 2) * 12)
    nb = jnp.where(small, 8, 12)
    base = h * 81
    L = _E * _E

    def it(i, carry):
        off = i * 16
        iv = idx_v[pl.ds(off, 16)] + base
        master_v[pl.ds(off, 16)] = plsc.load_gather(tab_v, [iv])
        return carry

    lax.fori_loop(0, _E * _MWP // 16, it, 0)

    for k in range(12):
        @pl.when(k < nb)
        def _():
            ri = rstart + k
            c0 = (_E - 1 - ri) * _E
            obase = (h * L + ri * _E) * L
            cps = [
                pltpu.async_copy(
                    master_v.at[pl.ds(ci * _MWP + c0, L)],
                    out_hbm.at[pl.ds(obase + ci * L, L)],
                    sem)
                for ci in range(_E)
            ]
            for cp in cps:
                cp.wait()


def kernel(x, lookup_table):
    L = x.shape[2]
    H, B = lookup_table.shape
    idx_const = jnp.asarray(_master_idx(L))
    tab_flat = lookup_table.reshape(H * B)

    mesh = plsc.VectorSubcoreMesh(core_axis_name="c", subcore_axis_name="s")
    run = pl.kernel(
        _sc_body,
        mesh=mesh,
        compiler_params=pltpu.CompilerParams(needs_layout_passes=False),
        out_type=jax.ShapeDtypeStruct((H * L * L,), jnp.float32),
        scratch_types=[
            pltpu.VMEM((H * B,), jnp.float32),
            pltpu.VMEM((_E * _MWP,), jnp.int32),
            pltpu.VMEM((_E * _MWP,), jnp.float32),
            pltpu.SemaphoreType.DMA,
        ],
    )
    out = run(tab_flat, idx_const)
    return out.reshape(1, H, L, L)
